# Initial kernel scaffold; baseline (speedup 1.0000x reference)
import jax
import jax.numpy as jnp
from jax.experimental import pallas as pl


def kernel(x, adj, W, a_src, a_dst):
    raise NotImplementedError("write your pallas kernel here")


# speedup vs baseline: 1.1494x; 1.1494x over previous
import functools

import jax
import jax.numpy as jnp
from jax.experimental import pallas as pl

B, N, IN_DIM = 8, 512, 64
HEADS, HEAD_DIM = 4, 16
NEG_INF = float("-inf")


def _gat_kernel(x_ref, adj_ref, w_ref, asrc_ref, adst_ref, out_ref):
    xb = x_ref[0]
    adjb = adj_ref[0]

    h = jnp.dot(xb, w_ref[...], preferred_element_type=jnp.float32)

    mask = adjb > 0.5
    has_edge = jnp.any(mask)
    row = jax.lax.broadcasted_iota(jnp.int32, (N, N), 0)
    col = jax.lax.broadcasted_iota(jnp.int32, (N, N), 1)
    mask = mask | ((row == col) & jnp.logical_not(has_edge))

    for hh in range(HEADS):
        h_head = h[:, hh * HEAD_DIM:(hh + 1) * HEAD_DIM]
        a_s = asrc_ref[hh:hh + 1, :]
        a_d = adst_ref[hh:hh + 1, :]
        s = jax.lax.dot_general(h_head, a_s, (((1,), (1,)), ((), ())),
                                preferred_element_type=jnp.float32)
        d = jax.lax.dot_general(a_d, h_head, (((1,), (1,)), ((), ())),
                                preferred_element_type=jnp.float32)
        e = s + d
        e = jnp.maximum(e, 0.2 * e)
        em = jnp.where(mask, e, NEG_INF)
        m = jnp.max(em, axis=0, keepdims=True)
        m = jnp.where(jnp.isfinite(m), m, 0.0)
        ex = jnp.exp(em - m)
        denom = jnp.sum(ex, axis=0, keepdims=True)
        alpha = ex * (1.0 / (denom + 1e-16))
        out = jax.lax.dot_general(alpha, h_head, (((0,), (0,)), ((), ())),
                                  preferred_element_type=jnp.float32)
        out_ref[0, :, hh * HEAD_DIM:(hh + 1) * HEAD_DIM] = out


@jax.jit
def kernel(x, adj, W, a_src, a_dst):
    w_flat = W.reshape(IN_DIM, HEADS * HEAD_DIM)
    grid = (B,)
    return pl.pallas_call(
        _gat_kernel,
        grid=grid,
        in_specs=[
            pl.BlockSpec((1, N, IN_DIM), lambda b: (b, 0, 0)),
            pl.BlockSpec((1, N, N), lambda b: (b, 0, 0)),
            pl.BlockSpec((IN_DIM, HEADS * HEAD_DIM), lambda b: (0, 0)),
            pl.BlockSpec((HEADS, HEAD_DIM), lambda b: (0, 0)),
            pl.BlockSpec((HEADS, HEAD_DIM), lambda b: (0, 0)),
        ],
        out_specs=pl.BlockSpec((1, N, HEADS * HEAD_DIM), lambda b: (b, 0, 0)),
        out_shape=jax.ShapeDtypeStruct((B, N, HEADS * HEAD_DIM), jnp.float32),
    )(x, adj, w_flat, a_src, a_dst)
